# baseline (speedup 1.0000x reference)
import functools

import jax
import jax.numpy as jnp
from jax import lax
from jax.experimental import pallas as pl
from jax.experimental.pallas import tpu as pltpu
from jax.experimental.pallas import tpu_sc as plsc

EMB = 256
NUM_E = 1024
N_ROWS = 8192
BLK = 1024
NSTEPS = N_ROWS // BLK
COMMIT = 0.25


def _tc_argmin_body(x_ref, wt_ref, idx_ref, m_ref, colsum_ref):
    step = pl.program_id(0)
    wt = wt_ref[...]

    @pl.when(step == 0)
    def _init():
        colsum_ref[...] = -0.5 * jnp.sum(wt, axis=0, keepdims=True)

    x = x_ref[...]
    mm = jnp.dot(x, wt, preferred_element_type=jnp.float32)
    rowsq = jnp.sum(x * x, axis=1, keepdims=True)
    dist = (rowsq + mm) + colsum_ref[...]
    m = jnp.min(dist, axis=1, keepdims=True)
    ids_f = lax.broadcasted_iota(jnp.int32, (1, NUM_E), 1).astype(jnp.float32)
    idxv_f = jnp.min(jnp.where(dist == m, ids_f, jnp.float32(NUM_E)),
                     axis=1, keepdims=True)
    idx_ref[...] = idxv_f.astype(jnp.int32).reshape(1, BLK, 1)
    m_ref[...] = m.reshape(1, BLK, 1)


_tc_argmin = pl.pallas_call(
    _tc_argmin_body,
    grid=(NSTEPS,),
    in_specs=[
        pl.BlockSpec((BLK, EMB), lambda i: (i, 0)),
        pl.BlockSpec((EMB, NUM_E), lambda i: (0, 0)),
    ],
    out_specs=[
        pl.BlockSpec((1, BLK, 1), lambda i: (i, 0, 0)),
        pl.BlockSpec((1, BLK, 1), lambda i: (i, 0, 0)),
    ],
    out_shape=[
        jax.ShapeDtypeStruct((NSTEPS, BLK, 1), jnp.int32),
        jax.ShapeDtypeStruct((NSTEPS, BLK, 1), jnp.float32),
    ],
    scratch_shapes=[
        pltpu.VMEM((1, NUM_E), jnp.float32),
    ],
)


def _tc_enc_body(idx_ref, m_ref, wt_ref, enc_ref, loss_ref, perp_ref,
                 delta_ref, counts_ref, acc_ref):
    step = pl.program_id(0)

    @pl.when(step == 0)
    def _init():
        wt = wt_ref[...]
        colsum = -0.5 * jnp.sum(wt, axis=0, keepdims=True)
        sqnorm = 0.25 * jnp.sum(wt * wt, axis=0, keepdims=True)
        delta_ref[...] = colsum - sqnorm
        counts_ref[...] = jnp.zeros_like(counts_ref)
        acc_ref[...] = jnp.zeros_like(acc_ref)

    idx_f = idx_ref[...].reshape(BLK, 1).astype(jnp.float32)
    m = m_ref[...].reshape(BLK, 1)
    ids_f = lax.broadcasted_iota(jnp.int32, (1, NUM_E), 1).astype(jnp.float32)
    enc = jnp.where(ids_f == idx_f, 1.0, 0.0).astype(jnp.float32)
    enc_ref[...] = enc
    cs = jnp.sum(enc, axis=0, keepdims=True)
    counts_ref[...] += cs
    acc_ref[...] += jnp.sum(m) - jnp.sum(cs * delta_ref[...])

    @pl.when(step == NSTEPS - 1)
    def _fin():
        loss_ref[...] = (COMMIT / jnp.float32(N_ROWS * EMB)) * acc_ref[...]
        avg = counts_ref[...] / jnp.float32(N_ROWS)
        ent = -jnp.sum(avg * jnp.log(avg + 1e-10), keepdims=True)
        perp_ref[...] = jnp.exp(ent).reshape(1, 1)


_tc_enc = pl.pallas_call(
    _tc_enc_body,
    grid=(NSTEPS,),
    in_specs=[
        pl.BlockSpec((1, BLK, 1), lambda i: (i, 0, 0)),
        pl.BlockSpec((1, BLK, 1), lambda i: (i, 0, 0)),
        pl.BlockSpec((EMB, NUM_E), lambda i: (0, 0)),
    ],
    out_specs=[
        pl.BlockSpec((BLK, NUM_E), lambda i: (i, 0)),
        pl.BlockSpec((1, 1), lambda i: (0, 0)),
        pl.BlockSpec((1, 1), lambda i: (0, 0)),
    ],
    out_shape=[
        jax.ShapeDtypeStruct((N_ROWS, NUM_E), jnp.float32),
        jax.ShapeDtypeStruct((1, 1), jnp.float32),
        jax.ShapeDtypeStruct((1, 1), jnp.float32),
    ],
    scratch_shapes=[
        pltpu.VMEM((1, NUM_E), jnp.float32),
        pltpu.VMEM((1, NUM_E), jnp.float32),
        pltpu.VMEM((1, 1), jnp.float32),
    ],
)

_SC_NUM_CORES = 2
_SC_NUM_SUBCORES = 16
_NW = _SC_NUM_CORES * _SC_NUM_SUBCORES
_ROWS_PER_W = N_ROWS // _NW


@functools.lru_cache(maxsize=1)
def _make_sc_gather():
    mesh = plsc.VectorSubcoreMesh(
        core_axis_name="c", subcore_axis_name="s",
        num_cores=_SC_NUM_CORES, num_subcores=_SC_NUM_SUBCORES)

    @functools.partial(
        pl.kernel,
        out_type=jax.ShapeDtypeStruct((N_ROWS, EMB), jnp.float32),
        mesh=mesh,
        scratch_types=[
            pltpu.VMEM((_ROWS_PER_W,), jnp.int32),
            pltpu.VMEM((_ROWS_PER_W, EMB), jnp.float32),
            pltpu.SemaphoreType.DMA,
        ],
    )
    def _sc_gather(table_hbm, idx_hbm, out_hbm, idx_v, rows_v, sem):
        wid = lax.axis_index("s") * _SC_NUM_CORES + lax.axis_index("c")
        base = wid * _ROWS_PER_W
        pltpu.sync_copy(idx_hbm.at[pl.ds(base, _ROWS_PER_W)], idx_v)
        pltpu.async_copy(table_hbm.at[idx_v], rows_v, sem).wait()
        pltpu.sync_copy(rows_v, out_hbm.at[pl.ds(base, _ROWS_PER_W)])

    return _sc_gather


def kernel(inputs, w):
    x2d = inputs.reshape(-1, EMB)
    wt = -2.0 * w.T
    idx3, m3 = _tc_argmin(x2d, wt)
    q = _make_sc_gather()(w, idx3.reshape(N_ROWS))
    enc, loss, perp = _tc_enc(idx3, m3, wt)
    quantized_st = q.reshape(inputs.shape)
    return (loss[0, 0], quantized_st, perp[0, 0], enc)


# speedup vs baseline: 1.1400x; 1.1400x over previous
import functools

import jax
import jax.numpy as jnp
from jax import lax
from jax.experimental import pallas as pl
from jax.experimental.pallas import tpu as pltpu
from jax.experimental.pallas import tpu_sc as plsc

EMB = 256
NUM_E = 1024
N_ROWS = 8192
BLK = 1024
NSTEPS = N_ROWS // BLK
COMMIT = 0.25


def _tc_body(x_ref, w_ref, enc_ref, idx_ref, loss_ref, perp_ref,
             wt_ref, colsum_ref, delta_ref, counts_ref, acc_ref):
    step = pl.program_id(0)

    @pl.when(step == 0)
    def _init():
        wtv = -2.0 * w_ref[...].T
        wt_ref[...] = wtv
        colsum = -0.5 * jnp.sum(wtv, axis=0, keepdims=True)
        sqnorm = 0.25 * jnp.sum(wtv * wtv, axis=0, keepdims=True)
        colsum_ref[...] = colsum
        delta_ref[...] = colsum - sqnorm
        counts_ref[...] = jnp.zeros_like(counts_ref)
        acc_ref[...] = jnp.zeros_like(acc_ref)

    wt = wt_ref[...]
    x = x_ref[...]
    mm = jnp.dot(x, wt, preferred_element_type=jnp.float32)
    rowsq = jnp.sum(x * x, axis=1, keepdims=True)
    dist = (rowsq + mm) + colsum_ref[...]
    m = jnp.min(dist, axis=1, keepdims=True)
    ids_f = lax.broadcasted_iota(jnp.int32, (1, NUM_E), 1).astype(jnp.float32)
    idxv_f = jnp.min(jnp.where(dist == m, ids_f, jnp.float32(NUM_E)),
                     axis=1, keepdims=True)
    enc = jnp.where(ids_f == idxv_f, 1.0, 0.0).astype(jnp.float32)
    enc_ref[...] = enc
    idx_ref[...] = idxv_f.astype(jnp.int32).reshape(1, BLK, 1)
    cs = jnp.sum(enc, axis=0, keepdims=True)
    counts_ref[...] += cs
    acc_ref[...] += jnp.sum(m) - jnp.sum(cs * delta_ref[...])

    @pl.when(step == NSTEPS - 1)
    def _fin():
        loss_ref[...] = (COMMIT / jnp.float32(N_ROWS * EMB)) * acc_ref[...]
        avg = counts_ref[...] / jnp.float32(N_ROWS)
        ent = -jnp.sum(avg * jnp.log(avg + 1e-10), keepdims=True)
        perp_ref[...] = jnp.exp(ent).reshape(1, 1)


_tc_call = pl.pallas_call(
    _tc_body,
    grid=(NSTEPS,),
    in_specs=[
        pl.BlockSpec((BLK, EMB), lambda i: (i, 0)),
        pl.BlockSpec((NUM_E, EMB), lambda i: (0, 0)),
    ],
    out_specs=[
        pl.BlockSpec((BLK, NUM_E), lambda i: (i, 0)),
        pl.BlockSpec((1, BLK, 1), lambda i: (i, 0, 0)),
        pl.BlockSpec((1, 1), lambda i: (0, 0)),
        pl.BlockSpec((1, 1), lambda i: (0, 0)),
    ],
    out_shape=[
        jax.ShapeDtypeStruct((N_ROWS, NUM_E), jnp.float32),
        jax.ShapeDtypeStruct((NSTEPS, BLK, 1), jnp.int32),
        jax.ShapeDtypeStruct((1, 1), jnp.float32),
        jax.ShapeDtypeStruct((1, 1), jnp.float32),
    ],
    scratch_shapes=[
        pltpu.VMEM((EMB, NUM_E), jnp.float32),
        pltpu.VMEM((1, NUM_E), jnp.float32),
        pltpu.VMEM((1, NUM_E), jnp.float32),
        pltpu.VMEM((1, NUM_E), jnp.float32),
        pltpu.VMEM((1, 1), jnp.float32),
    ],
)

_SC_NUM_CORES = 2
_SC_NUM_SUBCORES = 16
_NW = _SC_NUM_CORES * _SC_NUM_SUBCORES
_ROWS_PER_W = N_ROWS // _NW

_N_CHUNK = 2
_CH = _ROWS_PER_W // _N_CHUNK


@functools.lru_cache(maxsize=1)
def _make_sc_gather():
    mesh = plsc.VectorSubcoreMesh(
        core_axis_name="c", subcore_axis_name="s",
        num_cores=_SC_NUM_CORES, num_subcores=_SC_NUM_SUBCORES)

    @functools.partial(
        pl.kernel,
        out_type=jax.ShapeDtypeStruct((N_ROWS, EMB), jnp.float32),
        mesh=mesh,
        scratch_types=[
            pltpu.VMEM((_ROWS_PER_W,), jnp.int32),
            pltpu.VMEM((_ROWS_PER_W, EMB), jnp.float32),
            pltpu.SemaphoreType.DMA,
            pltpu.SemaphoreType.DMA,
        ],
    )
    def _sc_gather(table_hbm, idx_hbm, out_hbm, idx_v, rows_bufs, gsem, osem):
        wid = lax.axis_index("s") * _SC_NUM_CORES + lax.axis_index("c")
        base = wid * _ROWS_PER_W
        pltpu.sync_copy(idx_hbm.at[pl.ds(base, _ROWS_PER_W)], idx_v)
        pltpu.async_copy(table_hbm.at[idx_v], rows_bufs, gsem).wait()
        pltpu.sync_copy(rows_bufs, out_hbm.at[pl.ds(base, _ROWS_PER_W)])

    return _sc_gather


def kernel(inputs, w):
    x2d = inputs.reshape(-1, EMB)
    enc, idx3, loss, perp = _tc_call(x2d, w)
    idx = idx3.reshape(N_ROWS)
    q = _make_sc_gather()(w, idx)
    quantized_st = q.reshape(inputs.shape)
    return (loss[0, 0], quantized_st, perp[0, 0], enc)


# speedup vs baseline: 1.1734x; 1.0293x over previous
import functools

import jax
import jax.numpy as jnp
from jax import lax
from jax.experimental import pallas as pl
from jax.experimental.pallas import tpu as pltpu
from jax.experimental.pallas import tpu_sc as plsc

EMB = 256
NUM_E = 1024
N_ROWS = 8192
BLK = 2048
NSTEPS = N_ROWS // BLK
COMMIT = 0.25


def _tc_body(x_ref, w_ref, enc_ref, idx_ref, loss_ref, perp_ref,
             wt_ref, colsum_ref, delta_ref, counts_ref, acc_ref):
    step = pl.program_id(0)

    @pl.when(step == 0)
    def _init():
        wtv = -2.0 * w_ref[...].T
        wt_ref[...] = wtv
        colsum = -0.5 * jnp.sum(wtv, axis=0, keepdims=True)
        sqnorm = 0.25 * jnp.sum(wtv * wtv, axis=0, keepdims=True)
        colsum_ref[...] = colsum
        delta_ref[...] = colsum - sqnorm
        counts_ref[...] = jnp.zeros_like(counts_ref)
        acc_ref[...] = jnp.zeros_like(acc_ref)

    wt = wt_ref[...]
    x = x_ref[...]
    mm = jnp.dot(x, wt, preferred_element_type=jnp.float32)
    rowsq = jnp.sum(x * x, axis=1, keepdims=True)
    dist = (rowsq + mm) + colsum_ref[...]
    m = jnp.min(dist, axis=1, keepdims=True)
    ids_f = lax.broadcasted_iota(jnp.int32, (1, NUM_E), 1).astype(jnp.float32)
    idxv_f = jnp.min(jnp.where(dist == m, ids_f, jnp.float32(NUM_E)),
                     axis=1, keepdims=True)
    enc = jnp.where(ids_f == idxv_f, 1.0, 0.0).astype(jnp.float32)
    enc_ref[...] = enc
    idx_ref[...] = idxv_f.astype(jnp.int32).reshape(1, BLK, 1)
    cs = jnp.sum(enc, axis=0, keepdims=True)
    counts_ref[...] += cs
    acc_ref[...] += jnp.sum(m) - jnp.sum(cs * delta_ref[...])

    @pl.when(step == NSTEPS - 1)
    def _fin():
        loss_ref[...] = (COMMIT / jnp.float32(N_ROWS * EMB)) * acc_ref[...]
        avg = counts_ref[...] / jnp.float32(N_ROWS)
        ent = -jnp.sum(avg * jnp.log(avg + 1e-10), keepdims=True)
        perp_ref[...] = jnp.exp(ent).reshape(1, 1)


_tc_call = pl.pallas_call(
    _tc_body,
    grid=(NSTEPS,),
    in_specs=[
        pl.BlockSpec((BLK, EMB), lambda i: (i, 0)),
        pl.BlockSpec((NUM_E, EMB), lambda i: (0, 0)),
    ],
    out_specs=[
        pl.BlockSpec((BLK, NUM_E), lambda i: (i, 0)),
        pl.BlockSpec((1, BLK, 1), lambda i: (i, 0, 0)),
        pl.BlockSpec((1, 1), lambda i: (0, 0)),
        pl.BlockSpec((1, 1), lambda i: (0, 0)),
    ],
    out_shape=[
        jax.ShapeDtypeStruct((N_ROWS, NUM_E), jnp.float32),
        jax.ShapeDtypeStruct((NSTEPS, BLK, 1), jnp.int32),
        jax.ShapeDtypeStruct((1, 1), jnp.float32),
        jax.ShapeDtypeStruct((1, 1), jnp.float32),
    ],
    scratch_shapes=[
        pltpu.VMEM((EMB, NUM_E), jnp.float32),
        pltpu.VMEM((1, NUM_E), jnp.float32),
        pltpu.VMEM((1, NUM_E), jnp.float32),
        pltpu.VMEM((1, NUM_E), jnp.float32),
        pltpu.VMEM((1, 1), jnp.float32),
    ],
)

_SC_NUM_CORES = 2
_SC_NUM_SUBCORES = 16
_NW = _SC_NUM_CORES * _SC_NUM_SUBCORES
_ROWS_PER_W = N_ROWS // _NW

_N_CHUNK = 2
_CH = _ROWS_PER_W // _N_CHUNK


@functools.lru_cache(maxsize=1)
def _make_sc_gather():
    mesh = plsc.VectorSubcoreMesh(
        core_axis_name="c", subcore_axis_name="s",
        num_cores=_SC_NUM_CORES, num_subcores=_SC_NUM_SUBCORES)

    @functools.partial(
        pl.kernel,
        out_type=jax.ShapeDtypeStruct((N_ROWS, EMB), jnp.float32),
        mesh=mesh,
        scratch_types=[
            pltpu.VMEM((_ROWS_PER_W,), jnp.int32),
            pltpu.VMEM((_ROWS_PER_W, EMB), jnp.float32),
            pltpu.SemaphoreType.DMA,
            pltpu.SemaphoreType.DMA,
        ],
    )
    def _sc_gather(table_hbm, idx_hbm, out_hbm, idx_v, rows_bufs, gsem, osem):
        wid = lax.axis_index("s") * _SC_NUM_CORES + lax.axis_index("c")
        base = wid * _ROWS_PER_W
        pltpu.sync_copy(idx_hbm.at[pl.ds(base, _ROWS_PER_W)], idx_v)
        pltpu.async_copy(table_hbm.at[idx_v], rows_bufs, gsem).wait()
        pltpu.sync_copy(rows_bufs, out_hbm.at[pl.ds(base, _ROWS_PER_W)])

    return _sc_gather


def kernel(inputs, w):
    x2d = inputs.reshape(-1, EMB)
    enc, idx3, loss, perp = _tc_call(x2d, w)
    idx = idx3.reshape(N_ROWS)
    q = _make_sc_gather()(w, idx)
    quantized_st = q.reshape(inputs.shape)
    return (loss[0, 0], quantized_st, perp[0, 0], enc)
